# baseline (speedup 1.0000x reference)
import functools

import jax
import jax.numpy as jnp
from jax import lax
from jax.experimental import pallas as pl
from jax.experimental.pallas import tpu as pltpu
from jax.experimental.pallas import tpu_sc as plsc

B = 16384
D = 16
NC = 2
NS = 16
NW = NC * NS
BPW = B // NW
CH = 128
NCH = BPW // CH


def _sc_gather(i1g, i2g, emb3, emb):
    @functools.partial(
        pl.kernel,
        mesh=plsc.VectorSubcoreMesh(core_axis_name="c", subcore_axis_name="s"),
        compiler_params=pltpu.CompilerParams(use_tc_tiling_on_sc=False),
        out_type=[
            jax.ShapeDtypeStruct((B, D), jnp.float32),
            jax.ShapeDtypeStruct((B, D), jnp.float32),
        ],
        scratch_types=[
            pltpu.VMEM((NCH, CH), jnp.int32),
            pltpu.VMEM((NCH, CH), jnp.int32),
            pltpu.VMEM((BPW, D), jnp.float32),
            pltpu.VMEM((BPW, D), jnp.float32),
            pltpu.SemaphoreType.DMA,
        ],
    )
    def k(i1_hbm, i2_hbm, emb3_hbm, emb_hbm, o1_hbm, o2_hbm,
          idx1_v, idx2_v, rows1_v, rows2_v, sem):
        wid = lax.axis_index("s") * NC + lax.axis_index("c")
        base = wid * BPW
        pltpu.sync_copy(i1_hbm.at[wid], idx1_v)
        pltpu.sync_copy(i2_hbm.at[wid], idx2_v)
        copies = []
        for j in range(NCH):
            copies.append(pltpu.async_copy(
                emb3_hbm.at[idx1_v.at[j]], rows1_v.at[pl.ds(j * CH, CH)], sem))
            copies.append(pltpu.async_copy(
                emb_hbm.at[idx2_v.at[j]], rows2_v.at[pl.ds(j * CH, CH)], sem))
        for c in copies:
            c.wait()
        pltpu.sync_copy(rows1_v, o1_hbm.at[pl.ds(base, BPW)])
        pltpu.sync_copy(rows2_v, o2_hbm.at[pl.ds(base, BPW)])

    return k(i1g, i2g, emb3, emb)


def _mlp(e1, e2, xo, W1a, W1b, W1c, b1, W2, b2, W3, b3):
    bm = 2048
    grid = B // bm

    def body(e1_ref, e2_ref, xo_ref, w1a_ref, w1b_ref, w1c_ref, b1_ref,
             w2_ref, b2_ref, w3_ref, b3_ref, o_ref):
        h = (e1_ref[...] @ w1a_ref[...]
             + e2_ref[...] @ w1b_ref[...]
             + xo_ref[...] @ w1c_ref[...]
             + b1_ref[...])
        h = jnp.maximum(h, 0.0)
        h = jnp.maximum(h @ w2_ref[...] + b2_ref[...], 0.0)
        o_ref[...] = h @ w3_ref[...] + b3_ref[...]

    fixed = lambda *shape: pl.BlockSpec(shape, lambda i: (0,) * len(shape))
    return pl.pallas_call(
        body,
        grid=(grid,),
        in_specs=[
            pl.BlockSpec((bm, D), lambda i: (i, 0)),
            pl.BlockSpec((bm, D), lambda i: (i, 0)),
            pl.BlockSpec((bm, 64), lambda i: (i, 0)),
            fixed(D, 128),
            fixed(D, 128),
            fixed(64, 128),
            fixed(1, 128),
            fixed(128, 128),
            fixed(1, 128),
            fixed(128, 1),
            fixed(1, 1),
        ],
        out_specs=pl.BlockSpec((bm, 1), lambda i: (i, 0)),
        out_shape=jax.ShapeDtypeStruct((B, 1), jnp.float32),
    )(e1, e2, xo, W1a, W1b, W1c, b1, W2, b2, W3, b3)


def kernel(x, emb3, emb, W1, b1, W2, b2, W3, b3):
    i1 = x[:, 0].astype(jnp.int32).reshape(NW, NCH, CH)
    i2 = x[:, 1].astype(jnp.int32).reshape(NW, NCH, CH)
    xo = x[:, 2:]
    e1, e2 = _sc_gather(i1, i2, emb3, emb)
    return _mlp(e1, e2, xo,
                W1[:D], W1[D:2 * D], W1[2 * D:],
                b1.reshape(1, -1), W2, b2.reshape(1, -1),
                W3, b3.reshape(1, 1))
